# baseline (speedup 1.0000x reference)
import functools

import jax
import jax.numpy as jnp
from jax import lax
from jax.experimental import pallas as pl
from jax.experimental.pallas import tpu as pltpu
from jax.experimental.pallas import tpu_sc as plsc

B = 4096
MAXLEN = 50
A = 128
HNL = 2048
NC = 2
NS = 16
NW = NC * NS
BPW = B // NW
CHUNK = 128
CPW = B * MAXLEN // (CHUNK * NW)


def _sc_gather_pool(tag2d, seg2d, rating, category, zeros, emb_rating,
                    emb_category, emb_tag):
    mesh = plsc.VectorSubcoreMesh(core_axis_name="c", subcore_axis_name="s")
    f32 = jnp.float32

    @functools.partial(
        pl.kernel,
        out_type=(
            jax.ShapeDtypeStruct((B, A), f32),
            jax.ShapeDtypeStruct((B, A), f32),
            jax.ShapeDtypeStruct((B, A), f32),
        ),
        mesh=mesh,
        scratch_types=[
            pltpu.VMEM((CPW + 1, CHUNK), jnp.int32),
            pltpu.VMEM((CPW, CHUNK), jnp.int32),
            pltpu.VMEM((BPW,), jnp.int32),
            pltpu.VMEM((CHUNK, A), f32),
            pltpu.VMEM((CHUNK, A), f32),
            pltpu.VMEM((BPW, A), f32),
            pltpu.VMEM_SHARED((B, A), f32),
            pltpu.SemaphoreType.DMA,
        ],
    )
    def body(tag_ref, seg_ref, rat_ref, cat_ref, zero_ref,
             er_ref, ec_ref, et_ref,
             rat_out, cat_out, sum_out,
             idx_v, seg_v, bidx_v, rows0_v, rows1_v, emb_v, acc, sem):
        wid = lax.axis_index("c") * NS + lax.axis_index("s")
        base = wid * BPW

        pltpu.sync_copy(zero_ref, acc.at[pl.ds(base, BPW)])

        pltpu.sync_copy(tag_ref.at[wid], idx_v.at[pl.ds(0, CPW)])
        pltpu.sync_copy(seg_ref.at[wid], seg_v)
        for j in range(CHUNK // 16):
            idx_v[CPW, pl.ds(j * 16, 16)] = jnp.zeros((16,), jnp.int32)

        pltpu.sync_copy(rat_ref.at[pl.ds(base, BPW)], bidx_v)
        pltpu.async_copy(er_ref.at[bidx_v], emb_v, sem).wait()
        pltpu.sync_copy(emb_v, rat_out.at[pl.ds(base, BPW)])

        pltpu.sync_copy(cat_ref.at[pl.ds(base, BPW)], bidx_v)
        pltpu.async_copy(ec_ref.at[bidx_v], emb_v, sem).wait()
        pltpu.sync_copy(emb_v, cat_out.at[pl.ds(base, BPW)])

        bufs = (rows0_v, rows1_v)
        pltpu.async_copy(et_ref.at[idx_v.at[0]], bufs[0], sem)

        def step(s, carry):
            for b in range(2):
                k = 2 * s + b
                pltpu.make_async_copy(et_ref.at[idx_v.at[k]], bufs[b],
                                      sem).wait()
                nxt = jnp.minimum(k + 1, CPW)
                pltpu.async_copy(et_ref.at[idx_v.at[nxt]], bufs[1 - b], sem)
                pltpu.sync_copy(bufs[b], acc.at[seg_v.at[k]], add=True)
            return carry

        lax.fori_loop(0, CPW // 2, step, 0)
        pltpu.make_async_copy(et_ref.at[idx_v.at[0]], bufs[0], sem).wait()

        pltpu.sync_copy(acc.at[pl.ds(base, BPW)], sum_out.at[pl.ds(base, BPW)])

    return body(tag2d, seg2d, rating, category, zeros, emb_rating,
                emb_category, emb_tag)


def _tc_project(rat, cat, tsum, tag, w, b2d):
    BM = 512

    def body(tag_ref, rat_ref, cat_ref, tsum_ref, w_ref, b_ref,
             attr_ref, enc_ref):
        tl = jnp.sum((tag_ref[...] != 0).astype(jnp.float32), axis=1,
                     keepdims=True)
        tmean = tsum_ref[...] / tl
        x = jnp.concatenate([rat_ref[...], cat_ref[...], tmean], axis=1)
        attr_ref[...] = x
        y = lax.dot_general(x, w_ref[...], (((1,), (1,)), ((), ())),
                            preferred_element_type=jnp.float32)
        enc_ref[...] = jnp.tanh(y + b_ref[...])

    return pl.pallas_call(
        body,
        grid=(B // BM,),
        in_specs=[
            pl.BlockSpec((BM, MAXLEN), lambda i: (i, 0)),
            pl.BlockSpec((BM, A), lambda i: (i, 0)),
            pl.BlockSpec((BM, A), lambda i: (i, 0)),
            pl.BlockSpec((BM, A), lambda i: (i, 0)),
            pl.BlockSpec((HNL, 3 * A), lambda i: (0, 0)),
            pl.BlockSpec((1, HNL), lambda i: (0, 0)),
        ],
        out_specs=[
            pl.BlockSpec((BM, 3 * A), lambda i: (i, 0)),
            pl.BlockSpec((BM, HNL), lambda i: (i, 0)),
        ],
        out_shape=[
            jax.ShapeDtypeStruct((B, 3 * A), jnp.float32),
            jax.ShapeDtypeStruct((B, HNL), jnp.float32),
        ],
    )(tag, rat, cat, tsum, w, b2d)


def kernel(rating, category, tag, emb_rating, emb_category, emb_tag, W_out,
           b_out):
    rating_f = rating.reshape(B).astype(jnp.int32)
    category_f = category.reshape(B).astype(jnp.int32)
    tag_i = tag.astype(jnp.int32)
    tag2d = tag_i.reshape(NW, CPW, CHUNK)
    seg2d = jnp.repeat(jnp.arange(B, dtype=jnp.int32),
                       MAXLEN).reshape(NW, CPW, CHUNK)
    zeros = jnp.zeros((BPW, A), jnp.float32)
    rat_e, cat_e, tsum = _sc_gather_pool(tag2d, seg2d, rating_f, category_f,
                                         zeros, emb_rating, emb_category,
                                         emb_tag)
    attr, enc = _tc_project(rat_e, cat_e, tsum, tag_i, W_out,
                            b_out.reshape(1, HNL))
    return attr.reshape(B, 3, A), enc.reshape(B, 1, HNL)


# speedup vs baseline: 1.6744x; 1.6744x over previous
import functools

import jax
import jax.numpy as jnp
from jax import lax
from jax.experimental import pallas as pl
from jax.experimental.pallas import tpu as pltpu
from jax.experimental.pallas import tpu_sc as plsc

B = 4096
MAXLEN = 50
A = 128
HNL = 2048
NC = 2
NS = 16
NW = NC * NS
BPW = B // NW
CHUNK = 128
CPW = B * MAXLEN // (CHUNK * NW)


def _sc_gather_pool(tag2d, seg2d, rating, category, zeros, emb_rating,
                    emb_category, emb_tag):
    mesh = plsc.VectorSubcoreMesh(core_axis_name="c", subcore_axis_name="s")
    f32 = jnp.float32

    @functools.partial(
        pl.kernel,
        out_type=(
            jax.ShapeDtypeStruct((B, A), f32),
            jax.ShapeDtypeStruct((B, A), f32),
            jax.ShapeDtypeStruct((B, A), f32),
        ),
        mesh=mesh,
        scratch_types=[
            pltpu.VMEM((CPW, CHUNK), jnp.int32),
            pltpu.VMEM((CPW, CHUNK), jnp.int32),
            pltpu.VMEM((BPW,), jnp.int32),
            pltpu.VMEM((CHUNK, A), f32),
            pltpu.VMEM((CHUNK, A), f32),
            pltpu.VMEM((BPW, A), f32),
            pltpu.VMEM_SHARED((B, A), f32),
            pltpu.SemaphoreType.DMA,
            pltpu.SemaphoreType.DMA,
        ],
    )
    def body(tag_ref, seg_ref, rat_ref, cat_ref, zero_ref,
             er_ref, ec_ref, et_ref,
             rat_out, cat_out, sum_out,
             idx_v, seg_v, bidx_v, rows0_v, rows1_v, emb_v, acc, sem, sem2):
        wid = lax.axis_index("c") * NS + lax.axis_index("s")
        base = wid * BPW

        pltpu.sync_copy(zero_ref, acc.at[pl.ds(base, BPW)])

        pltpu.sync_copy(tag_ref.at[wid], idx_v)
        pltpu.sync_copy(seg_ref.at[wid], seg_v)

        pltpu.sync_copy(rat_ref.at[pl.ds(base, BPW)], bidx_v)
        pltpu.async_copy(er_ref.at[bidx_v], emb_v, sem).wait()
        pltpu.sync_copy(emb_v, rat_out.at[pl.ds(base, BPW)])

        pltpu.sync_copy(cat_ref.at[pl.ds(base, BPW)], bidx_v)
        pltpu.async_copy(ec_ref.at[bidx_v], emb_v, sem).wait()
        pltpu.sync_copy(emb_v, cat_out.at[pl.ds(base, BPW)])

        def step(s, carry):
            d0 = pltpu.async_copy(et_ref.at[idx_v.at[2 * s]], rows0_v, sem)
            d1 = pltpu.async_copy(et_ref.at[idx_v.at[2 * s + 1]], rows1_v,
                                  sem2)
            d0.wait()
            pltpu.sync_copy(rows0_v, acc.at[seg_v.at[2 * s]], add=True)
            d1.wait()
            pltpu.sync_copy(rows1_v, acc.at[seg_v.at[2 * s + 1]], add=True)
            return carry

        lax.fori_loop(0, CPW // 2, step, 0)

        pltpu.sync_copy(acc.at[pl.ds(base, BPW)], sum_out.at[pl.ds(base, BPW)])

    return body(tag2d, seg2d, rating, category, zeros, emb_rating,
                emb_category, emb_tag)


def _tc_project(rat, cat, tsum, tag, w, b2d):
    BM = 512

    def body(tag_ref, rat_ref, cat_ref, tsum_ref, w_ref, b_ref,
             attr_ref, enc_ref):
        tl = jnp.sum((tag_ref[...] != 0).astype(jnp.float32), axis=1,
                     keepdims=True)
        tmean = tsum_ref[...] / tl
        x = jnp.concatenate([rat_ref[...], cat_ref[...], tmean], axis=1)
        attr_ref[...] = x
        y = lax.dot_general(x, w_ref[...], (((1,), (1,)), ((), ())),
                            preferred_element_type=jnp.float32)
        enc_ref[...] = jnp.tanh(y + b_ref[...])

    return pl.pallas_call(
        body,
        grid=(B // BM,),
        in_specs=[
            pl.BlockSpec((BM, MAXLEN), lambda i: (i, 0)),
            pl.BlockSpec((BM, A), lambda i: (i, 0)),
            pl.BlockSpec((BM, A), lambda i: (i, 0)),
            pl.BlockSpec((BM, A), lambda i: (i, 0)),
            pl.BlockSpec((HNL, 3 * A), lambda i: (0, 0)),
            pl.BlockSpec((1, HNL), lambda i: (0, 0)),
        ],
        out_specs=[
            pl.BlockSpec((BM, 3 * A), lambda i: (i, 0)),
            pl.BlockSpec((BM, HNL), lambda i: (i, 0)),
        ],
        out_shape=[
            jax.ShapeDtypeStruct((B, 3 * A), jnp.float32),
            jax.ShapeDtypeStruct((B, HNL), jnp.float32),
        ],
    )(tag, rat, cat, tsum, w, b2d)


def kernel(rating, category, tag, emb_rating, emb_category, emb_tag, W_out,
           b_out):
    rating_f = rating.reshape(B).astype(jnp.int32)
    category_f = category.reshape(B).astype(jnp.int32)
    tag_i = tag.astype(jnp.int32)
    tag2d = tag_i.reshape(NW, CPW, CHUNK)
    seg2d = jnp.repeat(jnp.arange(B, dtype=jnp.int32),
                       MAXLEN).reshape(NW, CPW, CHUNK)
    zeros = jnp.zeros((BPW, A), jnp.float32)
    rat_e, cat_e, tsum = _sc_gather_pool(tag2d, seg2d, rating_f, category_f,
                                         zeros, emb_rating, emb_category,
                                         emb_tag)
    attr, enc = _tc_project(rat_e, cat_e, tsum, tag_i, W_out,
                            b_out.reshape(1, HNL))
    return attr.reshape(B, 3, A), enc.reshape(B, 1, HNL)


# speedup vs baseline: 1.7010x; 1.0159x over previous
import functools

import jax
import jax.numpy as jnp
from jax import lax
from jax.experimental import pallas as pl
from jax.experimental.pallas import tpu as pltpu
from jax.experimental.pallas import tpu_sc as plsc

B = 4096
MAXLEN = 50
A = 128
HNL = 2048
NC = 2
NS = 16
NW = NC * NS
BPW = B // NW
CHUNK = 128
CPW = B * MAXLEN // (CHUNK * NW)


def _sc_gather_pool(tag2d, seg2d, rating, category, zeros, emb_rating,
                    emb_category, emb_tag):
    mesh = plsc.VectorSubcoreMesh(core_axis_name="c", subcore_axis_name="s")
    f32 = jnp.float32

    @functools.partial(
        pl.kernel,
        out_type=(
            jax.ShapeDtypeStruct((B, A), f32),
            jax.ShapeDtypeStruct((B, A), f32),
            jax.ShapeDtypeStruct((B, A), f32),
        ),
        mesh=mesh,
        scratch_types=[
            pltpu.VMEM((CPW, CHUNK), jnp.int32),
            pltpu.VMEM((CPW, CHUNK), jnp.int32),
            pltpu.VMEM((BPW,), jnp.int32),
            pltpu.VMEM((CHUNK, A), f32),
            pltpu.VMEM((CHUNK, A), f32),
            pltpu.VMEM((CHUNK, A), f32),
            pltpu.VMEM((CHUNK, A), f32),
            pltpu.VMEM((BPW, A), f32),
            pltpu.VMEM_SHARED((B, A), f32),
            pltpu.SemaphoreType.DMA,
            pltpu.SemaphoreType.DMA,
            pltpu.SemaphoreType.DMA,
            pltpu.SemaphoreType.DMA,
        ],
    )
    def body(tag_ref, seg_ref, rat_ref, cat_ref, zero_ref,
             er_ref, ec_ref, et_ref,
             rat_out, cat_out, sum_out,
             idx_v, seg_v, bidx_v, rows0_v, rows1_v, rows2_v, rows3_v,
             emb_v, acc, sem0, sem1, sem2, sem3):
        wid = lax.axis_index("c") * NS + lax.axis_index("s")
        base = wid * BPW

        pltpu.sync_copy(zero_ref, acc.at[pl.ds(base, BPW)])

        pltpu.sync_copy(tag_ref.at[wid], idx_v)
        pltpu.sync_copy(seg_ref.at[wid], seg_v)

        pltpu.sync_copy(rat_ref.at[pl.ds(base, BPW)], bidx_v)
        pltpu.async_copy(er_ref.at[bidx_v], emb_v, sem0).wait()
        pltpu.sync_copy(emb_v, rat_out.at[pl.ds(base, BPW)])

        pltpu.sync_copy(cat_ref.at[pl.ds(base, BPW)], bidx_v)
        pltpu.async_copy(ec_ref.at[bidx_v], emb_v, sem0).wait()
        pltpu.sync_copy(emb_v, cat_out.at[pl.ds(base, BPW)])

        bufs = (rows0_v, rows1_v, rows2_v, rows3_v)
        sems = (sem0, sem1, sem2, sem3)
        nbuf = 4

        def step(s, carry):
            ds = [pltpu.async_copy(et_ref.at[idx_v.at[nbuf * s + b]],
                                   bufs[b], sems[b])
                  for b in range(nbuf)]
            for b in range(nbuf):
                ds[b].wait()
                pltpu.sync_copy(bufs[b], acc.at[seg_v.at[nbuf * s + b]],
                                add=True)
            return carry

        lax.fori_loop(0, CPW // nbuf, step, 0)

        tail = CPW % nbuf
        tds = [pltpu.async_copy(
                   et_ref.at[idx_v.at[CPW - tail + b]], bufs[b], sems[b])
               for b in range(tail)]
        for b in range(tail):
            tds[b].wait()
            pltpu.sync_copy(bufs[b], acc.at[seg_v.at[CPW - tail + b]],
                            add=True)

        pltpu.sync_copy(acc.at[pl.ds(base, BPW)], sum_out.at[pl.ds(base, BPW)])

    return body(tag2d, seg2d, rating, category, zeros, emb_rating,
                emb_category, emb_tag)


def _tc_project(rat, cat, tsum, tag, w, b2d):
    BM = 512

    def body(tag_ref, rat_ref, cat_ref, tsum_ref, w_ref, b_ref,
             attr_ref, enc_ref):
        tl = jnp.sum((tag_ref[...] != 0).astype(jnp.float32), axis=1,
                     keepdims=True)
        tmean = tsum_ref[...] / tl
        x = jnp.concatenate([rat_ref[...], cat_ref[...], tmean], axis=1)
        attr_ref[...] = x
        y = lax.dot_general(x, w_ref[...], (((1,), (1,)), ((), ())),
                            preferred_element_type=jnp.float32)
        enc_ref[...] = jnp.tanh(y + b_ref[...])

    return pl.pallas_call(
        body,
        grid=(B // BM,),
        in_specs=[
            pl.BlockSpec((BM, MAXLEN), lambda i: (i, 0)),
            pl.BlockSpec((BM, A), lambda i: (i, 0)),
            pl.BlockSpec((BM, A), lambda i: (i, 0)),
            pl.BlockSpec((BM, A), lambda i: (i, 0)),
            pl.BlockSpec((HNL, 3 * A), lambda i: (0, 0)),
            pl.BlockSpec((1, HNL), lambda i: (0, 0)),
        ],
        out_specs=[
            pl.BlockSpec((BM, 3 * A), lambda i: (i, 0)),
            pl.BlockSpec((BM, HNL), lambda i: (i, 0)),
        ],
        out_shape=[
            jax.ShapeDtypeStruct((B, 3 * A), jnp.float32),
            jax.ShapeDtypeStruct((B, HNL), jnp.float32),
        ],
    )(tag, rat, cat, tsum, w, b2d)


def kernel(rating, category, tag, emb_rating, emb_category, emb_tag, W_out,
           b_out):
    rating_f = rating.reshape(B).astype(jnp.int32)
    category_f = category.reshape(B).astype(jnp.int32)
    tag_i = tag.astype(jnp.int32)
    tag2d = tag_i.reshape(NW, CPW, CHUNK)
    seg2d = jnp.repeat(jnp.arange(B, dtype=jnp.int32),
                       MAXLEN).reshape(NW, CPW, CHUNK)
    zeros = jnp.zeros((BPW, A), jnp.float32)
    rat_e, cat_e, tsum = _sc_gather_pool(tag2d, seg2d, rating_f, category_f,
                                         zeros, emb_rating, emb_category,
                                         emb_tag)
    attr, enc = _tc_project(rat_e, cat_e, tsum, tag_i, W_out,
                            b_out.reshape(1, HNL))
    return attr.reshape(B, 3, A), enc.reshape(B, 1, HNL)


# speedup vs baseline: 1.7489x; 1.0282x over previous
import functools

import jax
import jax.numpy as jnp
from jax import lax
from jax.experimental import pallas as pl
from jax.experimental.pallas import tpu as pltpu
from jax.experimental.pallas import tpu_sc as plsc

B = 4096
MAXLEN = 50
A = 128
HNL = 2048
NC = 2
NS = 16
NW = NC * NS
BPW = B // NW
CHUNK = 128
CPW = B * MAXLEN // (CHUNK * NW)


def _sc_gather_pool(tag2d, seg2d, rating, category, zeros, emb_rating,
                    emb_category, emb_tag):
    mesh = plsc.VectorSubcoreMesh(core_axis_name="c", subcore_axis_name="s")
    f32 = jnp.float32

    @functools.partial(
        pl.kernel,
        out_type=(
            jax.ShapeDtypeStruct((B, A), f32),
            jax.ShapeDtypeStruct((B, A), f32),
            jax.ShapeDtypeStruct((B, A), f32),
        ),
        mesh=mesh,
        scratch_types=[
            pltpu.VMEM((CPW, CHUNK), jnp.int32),
            pltpu.VMEM((CPW, CHUNK), jnp.int32),
            pltpu.VMEM((BPW,), jnp.int32),
            pltpu.VMEM((BPW,), jnp.int32),
            pltpu.VMEM((CHUNK, A), f32),
            pltpu.VMEM((CHUNK, A), f32),
            pltpu.VMEM((CHUNK, A), f32),
            pltpu.VMEM((CHUNK, A), f32),
            pltpu.VMEM((BPW, A), f32),
            pltpu.VMEM((BPW, A), f32),
            pltpu.VMEM_SHARED((B // NC, A), f32),
            pltpu.SemaphoreType.DMA,
            pltpu.SemaphoreType.DMA,
            pltpu.SemaphoreType.DMA,
            pltpu.SemaphoreType.DMA,
            pltpu.SemaphoreType.DMA,
            pltpu.SemaphoreType.DMA,
        ],
    )
    def body(tag_ref, seg_ref, rat_ref, cat_ref, zero_ref,
             er_ref, ec_ref, et_ref,
             rat_out, cat_out, sum_out,
             idx_v, seg_v, ridx_v, cidx_v, rows0_v, rows1_v, rows2_v,
             rows3_v, remb_v, cemb_v, acc,
             sem0, sem1, sem2, sem3, semr, semc):
        wid = lax.axis_index("c") * NS + lax.axis_index("s")
        base = wid * BPW
        lbase = lax.axis_index("s") * BPW

        pltpu.sync_copy(zero_ref, acc.at[pl.ds(lbase, BPW)])

        pltpu.sync_copy(tag_ref.at[wid], idx_v)
        pltpu.sync_copy(seg_ref.at[wid], seg_v)

        pltpu.sync_copy(rat_ref.at[pl.ds(base, BPW)], ridx_v)
        pltpu.sync_copy(cat_ref.at[pl.ds(base, BPW)], cidx_v)
        rd = pltpu.async_copy(er_ref.at[ridx_v], remb_v, semr)
        cd = pltpu.async_copy(ec_ref.at[cidx_v], cemb_v, semc)

        bufs = (rows0_v, rows1_v, rows2_v, rows3_v)
        sems = (sem0, sem1, sem2, sem3)
        nbuf = 4

        def step(s, carry):
            ds = [pltpu.async_copy(et_ref.at[idx_v.at[nbuf * s + b]],
                                   bufs[b], sems[b])
                  for b in range(nbuf)]
            for b in range(nbuf):
                ds[b].wait()
                pltpu.sync_copy(bufs[b], acc.at[seg_v.at[nbuf * s + b]],
                                add=True)
            return carry

        lax.fori_loop(0, CPW // nbuf, step, 0)

        tail = CPW % nbuf
        tds = [pltpu.async_copy(
                   et_ref.at[idx_v.at[CPW - tail + b]], bufs[b], sems[b])
               for b in range(tail)]
        for b in range(tail):
            tds[b].wait()
            pltpu.sync_copy(bufs[b], acc.at[seg_v.at[CPW - tail + b]],
                            add=True)

        rd.wait()
        pltpu.sync_copy(remb_v, rat_out.at[pl.ds(base, BPW)])
        cd.wait()
        pltpu.sync_copy(cemb_v, cat_out.at[pl.ds(base, BPW)])
        pltpu.sync_copy(acc.at[pl.ds(lbase, BPW)],
                        sum_out.at[pl.ds(base, BPW)])

    return body(tag2d, seg2d, rating, category, zeros, emb_rating,
                emb_category, emb_tag)


def _tc_project(rat, cat, tsum, tag, w, b2d):
    BM = 512

    def body(tag_ref, rat_ref, cat_ref, tsum_ref, w_ref, b_ref,
             attr_ref, enc_ref):
        tl = jnp.sum((tag_ref[...] != 0).astype(jnp.float32), axis=1,
                     keepdims=True)
        tmean = tsum_ref[...] / tl
        x = jnp.concatenate([rat_ref[...], cat_ref[...], tmean], axis=1)
        attr_ref[...] = x
        y = lax.dot_general(x, w_ref[...], (((1,), (1,)), ((), ())),
                            preferred_element_type=jnp.float32)
        enc_ref[...] = jnp.tanh(y + b_ref[...])

    return pl.pallas_call(
        body,
        grid=(B // BM,),
        in_specs=[
            pl.BlockSpec((BM, MAXLEN), lambda i: (i, 0)),
            pl.BlockSpec((BM, A), lambda i: (i, 0)),
            pl.BlockSpec((BM, A), lambda i: (i, 0)),
            pl.BlockSpec((BM, A), lambda i: (i, 0)),
            pl.BlockSpec((HNL, 3 * A), lambda i: (0, 0)),
            pl.BlockSpec((1, HNL), lambda i: (0, 0)),
        ],
        out_specs=[
            pl.BlockSpec((BM, 3 * A), lambda i: (i, 0)),
            pl.BlockSpec((BM, HNL), lambda i: (i, 0)),
        ],
        out_shape=[
            jax.ShapeDtypeStruct((B, 3 * A), jnp.float32),
            jax.ShapeDtypeStruct((B, HNL), jnp.float32),
        ],
    )(tag, rat, cat, tsum, w, b2d)


def kernel(rating, category, tag, emb_rating, emb_category, emb_tag, W_out,
           b_out):
    rating_f = rating.reshape(B).astype(jnp.int32)
    category_f = category.reshape(B).astype(jnp.int32)
    tag_i = tag.astype(jnp.int32)
    tag2d = tag_i.reshape(NW, CPW, CHUNK)
    seg2d = (jnp.repeat(jnp.arange(B, dtype=jnp.int32), MAXLEN)
             % (B // NC)).reshape(NW, CPW, CHUNK)
    zeros = jnp.zeros((BPW, A), jnp.float32)
    rat_e, cat_e, tsum = _sc_gather_pool(tag2d, seg2d, rating_f, category_f,
                                         zeros, emb_rating, emb_category,
                                         emb_tag)
    attr, enc = _tc_project(rat_e, cat_e, tsum, tag_i, W_out,
                            b_out.reshape(1, HNL))
    return attr.reshape(B, 3, A), enc.reshape(B, 1, HNL)
